# Initial kernel scaffold; baseline (speedup 1.0000x reference)
import jax
import jax.numpy as jnp
from jax.experimental import pallas as pl


def kernel(iid, edge_index_interacts, agg_src, agg_dst, pid, tid, emb, pos_emb, tgt_emb, pi_w, q_w, r_w):
    raise NotImplementedError("write your pallas kernel here")


# speedup vs baseline: 3.0223x; 3.0223x over previous
import jax
import jax.numpy as jnp
from jax import lax
from jax.experimental import pallas as pl
from jax.experimental.pallas import tpu as pltpu
from jax.experimental.pallas import tpu_sc as plsc

DIM = 128
ALPHA = 0.2
NUM_NODE = 100000
N_ITEM = 10000
N_TGT = 1024
E_INT = 320000
E_AGG = 51200
NPOS = 200

NC, NS, L = 2, 16, 16
NW = NC * NS

_MESH = dict(core_axis_name="c", subcore_axis_name="s", num_cores=NC,
             num_subcores=NS)
_SC_PARAMS = pltpu.CompilerParams(needs_layout_passes=False)


def _wid():
    return lax.axis_index("s") * NC + lax.axis_index("c")


HN_PAD = 10240
HN_PER_W = HN_PAD // NW


def _hn_body(emb, iid, hn_out, idx_v, rows_v, sem):
    base = _wid() * HN_PER_W
    pltpu.sync_copy(iid.at[pl.ds(base, HN_PER_W)], idx_v)
    pltpu.async_copy(emb.at[idx_v], rows_v, sem).wait()
    pltpu.sync_copy(rows_v, hn_out.at[pl.ds(base, HN_PER_W)])


def _gather_hn(emb, iid_pad):
    return pl.kernel(
        _hn_body,
        out_type=jax.ShapeDtypeStruct((HN_PAD, DIM), jnp.float32),
        mesh=plsc.VectorSubcoreMesh(**_MESH),
        compiler_params=_SC_PARAMS,
        scratch_types=[
            pltpu.VMEM((HN_PER_W,), jnp.int32),
            pltpu.VMEM((HN_PER_W, DIM), jnp.float32),
            pltpu.SemaphoreType.DMA,
        ],
    )(emb, iid_pad)


EK = 80
E_PER_S = E_INT // NS
NCHUNK = E_PER_S // EK
N_ITEM_PAD = 10240
HALF = 5120
TRASH = HALF
HALF_PAD = HALF + 128
ZROWS_PER_S = HALF_PAD // NS
HROWS_PER_S = HALF // NS


def _edge_body(hn, src, dst, pw, ft_out,
               src_v, dst_v, ldst_v, hs_v, hd_v, w_v, pw_v,
               num_sp, den_sp, sem1, sem2):
    c = lax.axis_index("c")
    s = lax.axis_index("s")

    def zfill(i, carry):
        zv = jnp.zeros((L,), jnp.float32)
        for j in range(DIM // L):
            hd_v[i, pl.ds(j * L, L)] = zv
        return carry

    lax.fori_loop(0, EK, zfill, 0)
    for k, rows in ((0, 80), (80, 80), (160, 80), (240, 80), (320, 8)):
        pltpu.sync_copy(hd_v.at[pl.ds(0, rows)],
                        num_sp.at[pl.ds(s * ZROWS_PER_S + k, rows)])
        pltpu.sync_copy(hd_v.at[pl.ds(0, rows)],
                        den_sp.at[pl.ds(s * ZROWS_PER_S + k, rows)])
    pltpu.sync_copy(pw, pw_v)
    plsc.subcore_barrier()

    base0 = s * E_PER_S
    lo = c * HALF

    def chunk(ci, carry):
        base = base0 + ci * EK
        pltpu.sync_copy(src.at[pl.ds(base, EK)], src_v)
        pltpu.sync_copy(dst.at[pl.ds(base, EK)], dst_v)
        cp1 = pltpu.async_copy(hn.at[src_v], hs_v, sem1)
        cp2 = pltpu.async_copy(hn.at[dst_v], hd_v, sem2)
        cp1.wait()
        cp2.wait()
        for g in list(range(0, EK - 15, 16)) + ([EK - 16] if EK % 16 else []):
            d = dst_v[pl.ds(g, 16)]
            ld = d - lo
            ok = (ld >= 0) & (ld < HALF)
            ldst_v[pl.ds(g, 16)] = jnp.where(ok, ld, TRASH)
        pw_regs = [pw_v[pl.ds(j * L, L)] for j in range(DIM // L)]

        def edge(i, carry2):
            hs_regs = []
            acc = jnp.zeros((L,), jnp.float32)
            for j in range(DIM // L):
                a = hs_v[i, pl.ds(j * L, L)]
                b = hd_v[i, pl.ds(j * L, L)]
                hs_regs.append(a)
                acc = acc + a * b * pw_regs[j]
            e = jnp.sum(acc)
            ev = jnp.full((L,), e)
            ev = jnp.where(ev > 0, ev, ALPHA * ev)
            wv = jnp.exp(ev)
            for j in range(DIM // L):
                w_v[i, pl.ds(j * L, L)] = wv
            for j in range(DIM // L):
                hd_v[i, pl.ds(j * L, L)] = hs_regs[j] * wv
            return carry2

        lax.fori_loop(0, EK, edge, 0)
        pltpu.sync_copy(hd_v, num_sp.at[ldst_v], add=True)
        pltpu.sync_copy(w_v, den_sp.at[ldst_v], add=True)
        return carry

    lax.fori_loop(0, NCHUNK, chunk, 0)
    plsc.subcore_barrier()

    ro = s * HROWS_PER_S
    for p in range(HROWS_PER_S // EK):
        pltpu.sync_copy(num_sp.at[pl.ds(ro + p * EK, EK)],
                        hs_v.at[pl.ds(0, EK)])
        pltpu.sync_copy(den_sp.at[pl.ds(ro + p * EK, EK)],
                        w_v.at[pl.ds(0, EK)])

        def ftdiv(i, carry):
            dv = w_v[i, pl.ds(0, L)]
            rinv = jnp.full((L,), 1.0) / jnp.where(dv > 0, dv, 1.0)
            for j in range(DIM // L):
                hs_v[i, pl.ds(j * L, L)] = hs_v[i, pl.ds(j * L, L)] * rinv
            return carry

        lax.fori_loop(0, EK, ftdiv, 0)
        pltpu.sync_copy(hs_v.at[pl.ds(0, EK)],
                        ft_out.at[pl.ds(c * HALF + ro + p * EK, EK)])


def _edge_agg(hn, src, dst, pw):
    return pl.kernel(
        _edge_body,
        out_type=jax.ShapeDtypeStruct((N_ITEM_PAD, DIM), jnp.float32),
        mesh=plsc.VectorSubcoreMesh(**_MESH),
        compiler_params=_SC_PARAMS,
        scratch_types=[
            pltpu.VMEM((EK,), jnp.int32),
            pltpu.VMEM((EK,), jnp.int32),
            pltpu.VMEM((EK,), jnp.int32),
            pltpu.VMEM((EK, DIM), jnp.float32),
            pltpu.VMEM((EK, DIM), jnp.float32),
            pltpu.VMEM((EK, DIM), jnp.float32),
            pltpu.VMEM((DIM,), jnp.float32),
            pltpu.VMEM_SHARED((HALF_PAD, DIM), jnp.float32),
            pltpu.VMEM_SHARED((HALF_PAD, DIM), jnp.float32),
            pltpu.SemaphoreType.DMA,
            pltpu.SemaphoreType.DMA,
        ],
    )(hn, src, dst, pw)


AK = 200
A_PER_W = E_AGG // NW
ACH = A_PER_W // AK


def _agg_body(ft, asrc, eft_out, idx_v, rows_v, sem):
    base0 = _wid() * A_PER_W

    def chunk(ci, carry):
        base = base0 + ci * AK
        pltpu.sync_copy(asrc.at[pl.ds(base, AK)], idx_v)
        pltpu.async_copy(ft.at[idx_v], rows_v, sem).wait()
        pltpu.sync_copy(rows_v, eft_out.at[pl.ds(base, AK)])
        return carry

    lax.fori_loop(0, ACH, chunk, 0)


def _agg_gather(ft, asrc):
    return pl.kernel(
        _agg_body,
        out_type=jax.ShapeDtypeStruct((E_AGG, DIM), jnp.float32),
        mesh=plsc.VectorSubcoreMesh(**_MESH),
        compiler_params=_SC_PARAMS,
        scratch_types=[
            pltpu.VMEM((AK,), jnp.int32),
            pltpu.VMEM((AK, DIM), jnp.float32),
            pltpu.SemaphoreType.DMA,
        ],
    )(ft, asrc)


EB = 512
NEB = E_AGG // EB


def _sel_body(eft_ref, pid_ref, adst_ref, pos_ref, tgt_ref, qw_ref, rw_ref,
              out_ref, f_sc, msum_sc, cnt_sc):
    p = pl.program_id(0)
    j = pl.program_id(1)

    adst_blk = adst_ref[0, 0, :]
    eft = eft_ref[...]
    oneD = (adst_blk[:, None]
            == lax.broadcasted_iota(jnp.int32, (EB, N_TGT), 1)
            ).astype(jnp.float32)

    @pl.when(jnp.logical_and(p == 0, j == 0))
    def _():
        msum_sc[...] = jnp.zeros_like(msum_sc)
        cnt_sc[...] = jnp.zeros_like(cnt_sc)

    @pl.when(p == 0)
    def _():
        msum_sc[...] += lax.dot_general(oneD, eft, (((0,), (0,)), ((), ())),
                                        preferred_element_type=jnp.float32)
        cnt_sc[...] += lax.dot_general(oneD, jnp.ones((EB, 8), jnp.float32),
                                       (((0,), (0,)), ((), ())),
                                       preferred_element_type=jnp.float32)

    @pl.when(jnp.logical_and(p == 1, j == 0))
    def _():
        meanv = msum_sc[...] / jnp.maximum(cnt_sc[...][:, 0:1], 1.0)
        f = lax.dot_general(tgt_ref[...], rw_ref[:, :DIM],
                            (((1,), (1,)), ((), ())),
                            preferred_element_type=jnp.float32)
        f = f + lax.dot_general(meanv, rw_ref[:, DIM:],
                                (((1,), (1,)), ((), ())),
                                preferred_element_type=jnp.float32)
        f_sc[...] = f
        out_ref[...] = jnp.zeros_like(out_ref)

    @pl.when(p == 1)
    def _():
        pid_blk = pid_ref[0, 0, :]
        oneP = (pid_blk[:, None]
                == lax.broadcasted_iota(jnp.int32, (EB, NPOS), 1)
                ).astype(jnp.float32)
        posP = lax.dot_general(pos_ref[...], qw_ref[:, DIM:],
                               (((1,), (1,)), ((), ())),
                               preferred_element_type=jnp.float32)
        z = lax.dot_general(eft, qw_ref[:, :DIM], (((1,), (1,)), ((), ())),
                            preferred_element_type=jnp.float32)
        z = z + jnp.dot(oneP, posP, preferred_element_type=jnp.float32)
        e2 = jnp.tanh(z)
        f_e = jnp.dot(oneD, f_sc[...], preferred_element_type=jnp.float32)
        coef = jnp.sum(e2 * f_e, axis=1, keepdims=True)
        out_ref[...] += lax.dot_general(oneD, eft * coef,
                                        (((0,), (0,)), ((), ())),
                                        preferred_element_type=jnp.float32)


def _select_kernel(eft, pid3, adst3, pos_emb, tgt_emb, q_w, r_w):
    return pl.pallas_call(
        _sel_body,
        grid=(2, NEB),
        in_specs=[
            pl.BlockSpec((EB, DIM), lambda p, j: (j, 0)),
            pl.BlockSpec((1, 1, EB), lambda p, j: (j, 0, 0)),
            pl.BlockSpec((1, 1, EB), lambda p, j: (j, 0, 0)),
            pl.BlockSpec((NPOS, DIM), lambda p, j: (0, 0)),
            pl.BlockSpec((1, DIM), lambda p, j: (0, 0)),
            pl.BlockSpec((DIM, 2 * DIM), lambda p, j: (0, 0)),
            pl.BlockSpec((DIM, 2 * DIM), lambda p, j: (0, 0)),
        ],
        out_specs=pl.BlockSpec((N_TGT, DIM), lambda p, j: (0, 0)),
        out_shape=jax.ShapeDtypeStruct((N_TGT, DIM), jnp.float32),
        scratch_shapes=[
            pltpu.VMEM((N_TGT, DIM), jnp.float32),
            pltpu.VMEM((N_TGT, DIM), jnp.float32),
            pltpu.VMEM((N_TGT, 8), jnp.float32),
        ],
    )(eft, pid3, adst3, pos_emb, tgt_emb, q_w, r_w)


CB = 2048
NCB = (NUM_NODE - 1 + CB - 1) // CB


def _scores_body(sel_ref, b_ref, out_ref):
    out_ref[...] = lax.dot_general(sel_ref[...], b_ref[...],
                                   (((1,), (1,)), ((), ())),
                                   preferred_element_type=jnp.float32)


def _scores_kernel(select, b):
    return pl.pallas_call(
        _scores_body,
        grid=(NCB,),
        in_specs=[
            pl.BlockSpec((N_TGT, DIM), lambda j: (0, 0)),
            pl.BlockSpec((CB, DIM), lambda j: (j, 0)),
        ],
        out_specs=pl.BlockSpec((N_TGT, CB), lambda j: (0, j)),
        out_shape=jax.ShapeDtypeStruct((N_TGT, NUM_NODE - 1), jnp.float32),
    )(select, b)


def kernel(iid, edge_index_interacts, agg_src, agg_dst, pid, tid,
           emb, pos_emb, tgt_emb, pi_w, q_w, r_w):
    iid_pad = jnp.pad(iid, (0, HN_PAD - N_ITEM))
    src = edge_index_interacts[0]
    dst = edge_index_interacts[1]
    pw = jnp.reshape(pi_w, (DIM,))

    hn = _gather_hn(emb, iid_pad)
    ft = _edge_agg(hn, src, dst, pw)
    eft = _agg_gather(ft, agg_src)
    pid3 = pid.reshape(NEB, 1, EB)
    adst3 = agg_dst.reshape(NEB, 1, EB)
    select = _select_kernel(eft, pid3, adst3, pos_emb, tgt_emb, q_w, r_w)
    b = lax.slice(emb, (1, 0), (NUM_NODE, DIM))
    return _scores_kernel(select, b)


# speedup vs baseline: 4.3956x; 1.4544x over previous
import jax
import jax.numpy as jnp
from jax import lax
from jax.experimental import pallas as pl
from jax.experimental.pallas import tpu as pltpu
from jax.experimental.pallas import tpu_sc as plsc

DIM = 128
ALPHA = 0.2
NUM_NODE = 100000
N_ITEM = 10000
N_TGT = 1024
E_INT = 320000
E_AGG = 51200
NPOS = 200

NC, NS, L = 2, 16, 16
NW = NC * NS

_MESH = dict(core_axis_name="c", subcore_axis_name="s", num_cores=NC,
             num_subcores=NS)
_SC_PARAMS = pltpu.CompilerParams(needs_layout_passes=False)


def _wid():
    return lax.axis_index("s") * NC + lax.axis_index("c")


HN_PAD = 10240
HN_PER_W = HN_PAD // NW


def _hn_body(emb, iid, hn_out, idx_v, rows_v, sem):
    base = _wid() * HN_PER_W
    pltpu.sync_copy(iid.at[pl.ds(base, HN_PER_W)], idx_v)
    pltpu.async_copy(emb.at[idx_v], rows_v, sem).wait()
    pltpu.sync_copy(rows_v, hn_out.at[pl.ds(base, HN_PER_W)])


def _gather_hn(emb, iid_pad):
    return pl.kernel(
        _hn_body,
        out_type=jax.ShapeDtypeStruct((HN_PAD, DIM), jnp.float32),
        mesh=plsc.VectorSubcoreMesh(**_MESH),
        compiler_params=_SC_PARAMS,
        scratch_types=[
            pltpu.VMEM((HN_PER_W,), jnp.int32),
            pltpu.VMEM((HN_PER_W, DIM), jnp.float32),
            pltpu.SemaphoreType.DMA,
        ],
    )(emb, iid_pad)


EK = 40
E_PER_S = E_INT // NS
NCHUNK = E_PER_S // EK
NPAIR = NCHUNK // 2
N_ITEM_PAD = 10240
HALF = 5120
TRASH = HALF
HALF_PAD = HALF + 128
ZROWS_PER_S = HALF_PAD // NS
HROWS_PER_S = HALF // NS


def _edge_body(hn, src, dst, pw, ft_out,
               src_v0, dst_v0, ldst_v0, hs_v0, hd_v0, msg_v0, w_v0,
               src_v1, dst_v1, ldst_v1, hs_v1, hd_v1, msg_v1, w_v1,
               pw_v, num_sp, den_sp,
               sg1_0, sg2_0, ss_0, sg1_1, sg2_1, ss_1):
    c = lax.axis_index("c")
    s = lax.axis_index("s")
    srcs = (src_v0, src_v1)
    dsts = (dst_v0, dst_v1)
    ldsts = (ldst_v0, ldst_v1)
    hss = (hs_v0, hs_v1)
    hds = (hd_v0, hd_v1)
    msgs = (msg_v0, msg_v1)
    ws = (w_v0, w_v1)
    sg1 = (sg1_0, sg1_1)
    sg2 = (sg2_0, sg2_1)
    ss = (ss_0, ss_1)

    def zfill(i, carry):
        zv = jnp.zeros((L,), jnp.float32)
        for j in range(DIM // L):
            msg_v0[i, pl.ds(j * L, L)] = zv
        return carry

    lax.fori_loop(0, EK, zfill, 0)
    for k in range(0, 320, 40):
        pltpu.sync_copy(msg_v0.at[pl.ds(0, 40)],
                        num_sp.at[pl.ds(s * ZROWS_PER_S + k, 40)])
        pltpu.sync_copy(msg_v0.at[pl.ds(0, 40)],
                        den_sp.at[pl.ds(s * ZROWS_PER_S + k, 40)])
    pltpu.sync_copy(msg_v0.at[pl.ds(0, 8)],
                    num_sp.at[pl.ds(s * ZROWS_PER_S + 320, 8)])
    pltpu.sync_copy(msg_v0.at[pl.ds(0, 8)],
                    den_sp.at[pl.ds(s * ZROWS_PER_S + 320, 8)])
    pltpu.sync_copy(pw, pw_v)
    plsc.subcore_barrier()

    base0 = s * E_PER_S
    lo = c * HALF
    pw_regs = [pw_v[pl.ds(j * L, L)] for j in range(DIM // L)]

    def start_chunk(ci, P):
        base = base0 + ci * EK
        pltpu.sync_copy(src.at[pl.ds(base, EK)], srcs[P])
        pltpu.sync_copy(dst.at[pl.ds(base, EK)], dsts[P])
        pltpu.async_copy(hn.at[srcs[P]], hss[P], sg1[P])
        pltpu.async_copy(hn.at[dsts[P]], hds[P], sg2[P])

    def process(pi, P):
        pltpu.make_async_copy(hn.at[srcs[P]], hss[P], sg1[P]).wait()
        pltpu.make_async_copy(hn.at[dsts[P]], hds[P], sg2[P]).wait()

        @pl.when(pi > 0)
        def _():
            pltpu.make_async_copy(msgs[P], num_sp.at[ldsts[P]], ss[P]).wait()
            pltpu.make_async_copy(ws[P], den_sp.at[ldsts[P]], ss[P]).wait()

        for g in list(range(0, EK - 15, 16)) + ([EK - 16] if EK % 16 else []):
            d = dsts[P][pl.ds(g, 16)]
            ld = d - lo
            ok = (ld >= 0) & (ld < HALF)
            ldsts[P][pl.ds(g, 16)] = jnp.where(ok, ld, TRASH)

        def edge(i, carry2):
            hs_regs = []
            acc = jnp.zeros((L,), jnp.float32)
            for j in range(DIM // L):
                a = hss[P][i, pl.ds(j * L, L)]
                b = hds[P][i, pl.ds(j * L, L)]
                hs_regs.append(a)
                acc = acc + a * b * pw_regs[j]
            e = jnp.sum(acc)
            ev = jnp.full((L,), e)
            ev = jnp.where(ev > 0, ev, ALPHA * ev)
            wv = jnp.exp(ev)
            for j in range(DIM // L):
                ws[P][i, pl.ds(j * L, L)] = wv
                msgs[P][i, pl.ds(j * L, L)] = hs_regs[j] * wv
            return carry2

        lax.fori_loop(0, EK, edge, 0)
        pltpu.async_copy(msgs[P], num_sp.at[ldsts[P]], ss[P], add=True)
        pltpu.async_copy(ws[P], den_sp.at[ldsts[P]], ss[P], add=True)

    start_chunk(0, 0)

    def pair(pi, carry):
        c0 = pi * 2
        start_chunk(c0 + 1, 1)
        process(pi, 0)

        @pl.when(pi < NPAIR - 1)
        def _():
            start_chunk(c0 + 2, 0)

        process(pi, 1)
        return carry

    lax.fori_loop(0, NPAIR, pair, 0)
    for P in range(2):
        pltpu.make_async_copy(msgs[P], num_sp.at[ldsts[P]], ss[P]).wait()
        pltpu.make_async_copy(ws[P], den_sp.at[ldsts[P]], ss[P]).wait()
    plsc.subcore_barrier()

    ro = s * HROWS_PER_S
    for p in range(HROWS_PER_S // EK):
        pltpu.sync_copy(num_sp.at[pl.ds(ro + p * EK, EK)],
                        hs_v0.at[pl.ds(0, EK)])
        pltpu.sync_copy(den_sp.at[pl.ds(ro + p * EK, EK)],
                        w_v0.at[pl.ds(0, EK)])

        def ftdiv(i, carry):
            dv = w_v0[i, pl.ds(0, L)]
            rinv = jnp.full((L,), 1.0) / jnp.where(dv > 0, dv, 1.0)
            for j in range(DIM // L):
                hs_v0[i, pl.ds(j * L, L)] = hs_v0[i, pl.ds(j * L, L)] * rinv
            return carry

        lax.fori_loop(0, EK, ftdiv, 0)
        pltpu.sync_copy(hs_v0.at[pl.ds(0, EK)],
                        ft_out.at[pl.ds(c * HALF + ro + p * EK, EK)])


def _edge_agg(hn, src, dst, pw):
    per_parity = [
        pltpu.VMEM((EK,), jnp.int32),
        pltpu.VMEM((EK,), jnp.int32),
        pltpu.VMEM((EK,), jnp.int32),
        pltpu.VMEM((EK, DIM), jnp.float32),
        pltpu.VMEM((EK, DIM), jnp.float32),
        pltpu.VMEM((EK, DIM), jnp.float32),
        pltpu.VMEM((EK, DIM), jnp.float32),
    ]
    return pl.kernel(
        _edge_body,
        out_type=jax.ShapeDtypeStruct((N_ITEM_PAD, DIM), jnp.float32),
        mesh=plsc.VectorSubcoreMesh(**_MESH),
        compiler_params=_SC_PARAMS,
        scratch_types=(per_parity + per_parity
                       + [pltpu.VMEM((DIM,), jnp.float32),
                          pltpu.VMEM_SHARED((HALF_PAD, DIM), jnp.float32),
                          pltpu.VMEM_SHARED((HALF_PAD, DIM), jnp.float32)]
                       + [pltpu.SemaphoreType.DMA] * 6),
    )(hn, src, dst, pw)


AK = 200
A_PER_W = E_AGG // NW
ACH = A_PER_W // AK


def _agg_body(ft, asrc, eft_out, idx_v, rows_v, sem):
    base0 = _wid() * A_PER_W

    def chunk(ci, carry):
        base = base0 + ci * AK
        pltpu.sync_copy(asrc.at[pl.ds(base, AK)], idx_v)
        pltpu.async_copy(ft.at[idx_v], rows_v, sem).wait()
        pltpu.sync_copy(rows_v, eft_out.at[pl.ds(base, AK)])
        return carry

    lax.fori_loop(0, ACH, chunk, 0)


def _agg_gather(ft, asrc):
    return pl.kernel(
        _agg_body,
        out_type=jax.ShapeDtypeStruct((E_AGG, DIM), jnp.float32),
        mesh=plsc.VectorSubcoreMesh(**_MESH),
        compiler_params=_SC_PARAMS,
        scratch_types=[
            pltpu.VMEM((AK,), jnp.int32),
            pltpu.VMEM((AK, DIM), jnp.float32),
            pltpu.SemaphoreType.DMA,
        ],
    )(ft, asrc)


EB = 512
NEB = E_AGG // EB


def _sel_body(eft_ref, pid_ref, adst_ref, pos_ref, tgt_ref, qw_ref, rw_ref,
              out_ref, f_sc, msum_sc, cnt_sc):
    p = pl.program_id(0)
    j = pl.program_id(1)

    adst_blk = adst_ref[0, 0, :]
    eft = eft_ref[...]
    oneD = (adst_blk[:, None]
            == lax.broadcasted_iota(jnp.int32, (EB, N_TGT), 1)
            ).astype(jnp.float32)

    @pl.when(jnp.logical_and(p == 0, j == 0))
    def _():
        msum_sc[...] = jnp.zeros_like(msum_sc)
        cnt_sc[...] = jnp.zeros_like(cnt_sc)

    @pl.when(p == 0)
    def _():
        msum_sc[...] += lax.dot_general(oneD, eft, (((0,), (0,)), ((), ())),
                                        preferred_element_type=jnp.float32)
        cnt_sc[...] += lax.dot_general(oneD, jnp.ones((EB, 8), jnp.float32),
                                       (((0,), (0,)), ((), ())),
                                       preferred_element_type=jnp.float32)

    @pl.when(jnp.logical_and(p == 1, j == 0))
    def _():
        meanv = msum_sc[...] / jnp.maximum(cnt_sc[...][:, 0:1], 1.0)
        f = lax.dot_general(tgt_ref[...], rw_ref[:, :DIM],
                            (((1,), (1,)), ((), ())),
                            preferred_element_type=jnp.float32)
        f = f + lax.dot_general(meanv, rw_ref[:, DIM:],
                                (((1,), (1,)), ((), ())),
                                preferred_element_type=jnp.float32)
        f_sc[...] = f
        out_ref[...] = jnp.zeros_like(out_ref)

    @pl.when(p == 1)
    def _():
        pid_blk = pid_ref[0, 0, :]
        oneP = (pid_blk[:, None]
                == lax.broadcasted_iota(jnp.int32, (EB, NPOS), 1)
                ).astype(jnp.float32)
        posP = lax.dot_general(pos_ref[...], qw_ref[:, DIM:],
                               (((1,), (1,)), ((), ())),
                               preferred_element_type=jnp.float32)
        z = lax.dot_general(eft, qw_ref[:, :DIM], (((1,), (1,)), ((), ())),
                            preferred_element_type=jnp.float32)
        z = z + jnp.dot(oneP, posP, preferred_element_type=jnp.float32)
        e2 = jnp.tanh(z)
        f_e = jnp.dot(oneD, f_sc[...], preferred_element_type=jnp.float32)
        coef = jnp.sum(e2 * f_e, axis=1, keepdims=True)
        out_ref[...] += lax.dot_general(oneD, eft * coef,
                                        (((0,), (0,)), ((), ())),
                                        preferred_element_type=jnp.float32)


def _select_kernel(eft, pid3, adst3, pos_emb, tgt_emb, q_w, r_w):
    return pl.pallas_call(
        _sel_body,
        grid=(2, NEB),
        in_specs=[
            pl.BlockSpec((EB, DIM), lambda p, j: (j, 0)),
            pl.BlockSpec((1, 1, EB), lambda p, j: (j, 0, 0)),
            pl.BlockSpec((1, 1, EB), lambda p, j: (j, 0, 0)),
            pl.BlockSpec((NPOS, DIM), lambda p, j: (0, 0)),
            pl.BlockSpec((1, DIM), lambda p, j: (0, 0)),
            pl.BlockSpec((DIM, 2 * DIM), lambda p, j: (0, 0)),
            pl.BlockSpec((DIM, 2 * DIM), lambda p, j: (0, 0)),
        ],
        out_specs=pl.BlockSpec((N_TGT, DIM), lambda p, j: (0, 0)),
        out_shape=jax.ShapeDtypeStruct((N_TGT, DIM), jnp.float32),
        scratch_shapes=[
            pltpu.VMEM((N_TGT, DIM), jnp.float32),
            pltpu.VMEM((N_TGT, DIM), jnp.float32),
            pltpu.VMEM((N_TGT, 8), jnp.float32),
        ],
    )(eft, pid3, adst3, pos_emb, tgt_emb, q_w, r_w)


CB = 2048
NCB = (NUM_NODE - 1 + CB - 1) // CB


def _scores_body(sel_ref, b_ref, out_ref):
    out_ref[...] = lax.dot_general(sel_ref[...], b_ref[...],
                                   (((1,), (1,)), ((), ())),
                                   preferred_element_type=jnp.float32)


def _scores_kernel(select, b):
    return pl.pallas_call(
        _scores_body,
        grid=(NCB,),
        in_specs=[
            pl.BlockSpec((N_TGT, DIM), lambda j: (0, 0)),
            pl.BlockSpec((CB, DIM), lambda j: (j, 0)),
        ],
        out_specs=pl.BlockSpec((N_TGT, CB), lambda j: (0, j)),
        out_shape=jax.ShapeDtypeStruct((N_TGT, NUM_NODE - 1), jnp.float32),
    )(select, b)


def kernel(iid, edge_index_interacts, agg_src, agg_dst, pid, tid,
           emb, pos_emb, tgt_emb, pi_w, q_w, r_w):
    iid_pad = jnp.pad(iid, (0, HN_PAD - N_ITEM))
    src = edge_index_interacts[0]
    dst = edge_index_interacts[1]
    pw = jnp.reshape(pi_w, (DIM,))

    hn = _gather_hn(emb, iid_pad)
    ft = _edge_agg(hn, src, dst, pw)
    eft = _agg_gather(ft, agg_src)
    pid3 = pid.reshape(NEB, 1, EB)
    adst3 = agg_dst.reshape(NEB, 1, EB)
    select = _select_kernel(eft, pid3, adst3, pos_emb, tgt_emb, q_w, r_w)
    b = lax.slice(emb, (1, 0), (NUM_NODE, DIM))
    return _scores_kernel(select, b)
